# baseline (speedup 1.0000x reference)
import functools

import jax
import jax.numpy as jnp
from jax import lax
from jax.experimental import pallas as pl
from jax.experimental.pallas import tpu as pltpu
from jax.experimental.pallas import tpu_sc as plsc

N = 100000
F = 128
G = 1024
SEG_PER_W = 32
C = 128
RB = 2000
NB = N // RB


def _score_body(x_ref, w1_ref, b1_ref, w2_ref, b2_ref, batch_ref,
                o_ref, st_ref, cacc):
    i = pl.program_id(0)

    @pl.when(i == 0)
    def _():
        cacc[...] = jnp.zeros_like(cacc)

    h = jnp.dot(x_ref[...], w1_ref[...], preferred_element_type=jnp.float32)
    h = h + b1_ref[...]
    h = jnp.maximum(h, 0.01 * h)
    s = jax.lax.dot_general(w2_ref[...], h, (((1,), (1,)), ((), ())),
                            preferred_element_type=jnp.float32)
    o_ref[0, 0, :] = s[0, :] + b2_ref[0, 0]

    bi = batch_ref[0, 0, :]
    hi_oh = jnp.equal(
        (bi // 128)[None, :],
        jax.lax.broadcasted_iota(jnp.int32, (8, RB), 0)).astype(jnp.float32)
    lo_oh = jnp.equal(
        (bi % 128)[:, None],
        jax.lax.broadcasted_iota(jnp.int32, (RB, F), 1)).astype(jnp.float32)
    cacc[...] += jnp.dot(hi_oh, lo_oh, preferred_element_type=jnp.float32)

    st_ref[...] = cacc[...].astype(jnp.int32)


def _score(x, W1, b1r, W2r, b2r, batch3d):
    return pl.pallas_call(
        _score_body,
        grid=(NB,),
        in_specs=[
            pl.BlockSpec((RB, F), lambda i: (i, 0)),
            pl.BlockSpec((F, F), lambda i: (0, 0)),
            pl.BlockSpec((1, F), lambda i: (0, 0)),
            pl.BlockSpec((1, F), lambda i: (0, 0)),
            pl.BlockSpec((1, 1), lambda i: (0, 0)),
            pl.BlockSpec((1, 1, RB), lambda i: (i, 0, 0)),
        ],
        out_specs=[
            pl.BlockSpec((1, 1, RB), lambda i: (i, 0, 0)),
            pl.BlockSpec((8, F), lambda i: (0, 0)),
        ],
        out_shape=[
            jax.ShapeDtypeStruct((NB, 1, RB), jnp.float32),
            jax.ShapeDtypeStruct((8, F), jnp.int32),
        ],
        scratch_shapes=[pltpu.VMEM((8, F), jnp.float32)],
    )(x, W1, b1r, W2r, b2r, batch3d)


def _sc_body(x_hbm, score_hbm, starts_hbm,
             maxp_hbm, sump_hbm, va_hbm, cnt_hbm, d_hbm,
             starts_v, xbuf, sbuf, stmax, stsum, stva, stcnt, std,
             semx0, semx1, sems0, sems1):
    wid = lax.axis_index("s") * 2 + lax.axis_index("c")
    seg_lo = wid * SEG_PER_W
    pltpu.sync_copy(starts_hbm.at[pl.ds(seg_lo, 64)], starts_v)

    neg_inf = jnp.full((16,), -jnp.inf, jnp.float32)
    zeros = jnp.zeros((16,), jnp.float32)
    init_acc = ((neg_inf,) * 8, (zeros,) * 8, (zeros,) * 8,
                jnp.float32(-jnp.inf), zeros)

    sva = starts_v[pl.ds(0, 16)]
    rs = sva[0]
    svb = starts_v[pl.ds(32, 16)]
    re = svb[0]
    rsa = (rs // 8) * 8
    nch = jnp.where(re > rs, (re - rsa + C - 1) // C, 0)

    def start_chunk(c):
        bb = jnp.minimum(rsa + c * C, N - C)

        @pl.when((c & 1) == 0)
        def _():
            pltpu.async_copy(x_hbm.at[pl.ds(bb, C)],
                             xbuf.at[pl.ds(0, C)], semx0)
            pltpu.async_copy(score_hbm.at[pl.ds(bb, C)],
                             sbuf.at[pl.ds(0, C)], sems0)

        @pl.when((c & 1) == 1)
        def _():
            pltpu.async_copy(x_hbm.at[pl.ds(bb, C)],
                             xbuf.at[pl.ds(C, C)], semx1)
            pltpu.async_copy(score_hbm.at[pl.ds(bb, C)],
                             sbuf.at[pl.ds(C, C)], sems1)

    def wait_chunk(c):
        @pl.when((c & 1) == 0)
        def _():
            pltpu.make_async_copy(x_hbm.at[pl.ds(0, C)],
                                  xbuf.at[pl.ds(0, C)], semx0).wait()
            pltpu.make_async_copy(score_hbm.at[pl.ds(0, C)],
                                  sbuf.at[pl.ds(0, C)], sems0).wait()

        @pl.when((c & 1) == 1)
        def _():
            pltpu.make_async_copy(x_hbm.at[pl.ds(0, C)],
                                  xbuf.at[pl.ds(C, C)], semx1).wait()
            pltpu.make_async_copy(score_hbm.at[pl.ds(0, C)],
                                  sbuf.at[pl.ds(C, C)], sems1).wait()

    @pl.when(nch > 0)
    def _():
        start_chunk(jnp.int32(0))

    def smax_body(r, ml):
        return jnp.maximum(ml, sbuf[pl.ds(r, 16)][0])

    def seg_body(s_rel, wdone):
        sv = starts_v[pl.ds(s_rel, 16)]
        s0 = sv[0]
        s1 = sv[1]
        cs0 = (s0 - rsa) // C
        cs1 = (s1 - 1 - rsa) // C
        ncs = jnp.where(s1 > s0, cs1 - cs0 + 1, 0)

        def piece(j, cy):
            acc, wdone = cy
            c = cs0 + j
            need = c >= wdone

            @pl.when(jnp.logical_and(need, c + 1 < nch))
            def _():
                start_chunk(c + 1)

            @pl.when(need)
            def _():
                wait_chunk(c)

            wdone = jnp.where(need, c + 1, wdone)
            base = rsa + c * C
            bb = jnp.minimum(base, N - C)
            off = (c & 1) * C - bb
            pg0 = jnp.maximum(s0, base)
            pg1 = jnp.minimum(s1, base + C)
            vmax, vsum, va, m, d = acc
            ml = lax.fori_loop(pg0 + off, pg1 + off, smax_body,
                               -jnp.inf)

            def row_body(r, pacc):
                vmax, vsum, vap, dp = pacc
                en = jnp.exp(
                    jnp.full((16,), sbuf[pl.ds(r, 16)][0] - ml, jnp.float32))
                xs = [xbuf[r, pl.ds(16 * k, 16)] for k in range(8)]
                vmax2 = tuple(jnp.maximum(vmax[k], xs[k]) for k in range(8))
                vsum2 = tuple(vsum[k] + xs[k] for k in range(8))
                vap2 = tuple(vap[k] + xs[k] * en for k in range(8))
                return (vmax2, vsum2, vap2, dp + en)

            zv = jnp.zeros((16,), jnp.float32)
            vmax, vsum, vap, dp = lax.fori_loop(
                pg0 + off, pg1 + off, row_body,
                (vmax, vsum, (zv,) * 8, zv))

            m2 = jnp.maximum(m, ml)
            ae = jnp.exp(jnp.full(
                (16,), jnp.where(m == m2, 0.0, m - m2), jnp.float32))
            pe = jnp.exp(jnp.full(
                (16,), jnp.where(ml == m2, 0.0, ml - m2), jnp.float32))
            d = d * ae + dp * pe
            va = tuple(va[k] * ae + vap[k] * pe for k in range(8))
            acc = (vmax, vsum, va, m2, d)
            return (acc, wdone)

        acc, wdone = lax.fori_loop(0, ncs, piece, (init_acc, wdone))
        vmax, vsum, va, m, d = acc
        cv = jnp.full((16,), (s1 - s0).astype(jnp.float32), jnp.float32)
        for k in range(8):
            stmax[pl.ds(s_rel * F + 16 * k, 16)] = vmax[k]
            stsum[pl.ds(s_rel * F + 16 * k, 16)] = vsum[k]
            stva[pl.ds(s_rel * F + 16 * k, 16)] = va[k]
        stcnt[pl.ds(s_rel * 16, 16)] = cv
        std[pl.ds(s_rel * 16, 16)] = d
        return wdone

    lax.fori_loop(0, SEG_PER_W, seg_body, 0)

    pltpu.sync_copy(stmax, maxp_hbm.at[pl.ds(seg_lo * F, SEG_PER_W * F)])
    pltpu.sync_copy(stsum, sump_hbm.at[pl.ds(seg_lo * F, SEG_PER_W * F)])
    pltpu.sync_copy(stva, va_hbm.at[pl.ds(seg_lo * F, SEG_PER_W * F)])
    pltpu.sync_copy(stcnt, cnt_hbm.at[pl.ds(seg_lo * 16, SEG_PER_W * 16)])
    pltpu.sync_copy(std, d_hbm.at[pl.ds(seg_lo * 16, SEG_PER_W * 16)])


def _sc_reduce(x, score, starts):
    mesh = plsc.VectorSubcoreMesh(core_axis_name="c", subcore_axis_name="s")
    f32 = jnp.float32
    fn = functools.partial(
        pl.kernel,
        mesh=mesh,
        out_type=[
            jax.ShapeDtypeStruct((G * F,), f32),
            jax.ShapeDtypeStruct((G * F,), f32),
            jax.ShapeDtypeStruct((G * F,), f32),
            jax.ShapeDtypeStruct((G * 16,), f32),
            jax.ShapeDtypeStruct((G * 16,), f32),
        ],
        scratch_types=[
            pltpu.VMEM((64,), jnp.int32),
            pltpu.VMEM((2 * C, F), f32),
            pltpu.VMEM((2 * C + 16,), f32),
            pltpu.VMEM((SEG_PER_W * F,), f32),
            pltpu.VMEM((SEG_PER_W * F,), f32),
            pltpu.VMEM((SEG_PER_W * F,), f32),
            pltpu.VMEM((SEG_PER_W * 16,), f32),
            pltpu.VMEM((SEG_PER_W * 16,), f32),
            pltpu.SemaphoreType.DMA,
            pltpu.SemaphoreType.DMA,
            pltpu.SemaphoreType.DMA,
            pltpu.SemaphoreType.DMA,
        ],
    )(_sc_body)
    maxp, sump, va, cnt, d = fn(x, score, starts)
    return (maxp.reshape(G, F), sump.reshape(G, F), va.reshape(G, F),
            cnt.reshape(G, 16), d.reshape(G, 16))


def _comb_body(maxp_ref, sump_ref, va_ref, cnt_ref, d_ref, wfc_ref, bfc_ref,
               o_ref):
    cnt = cnt_ref[...][:, 0:1]
    den = d_ref[...][:, 0:1]
    ok = cnt > 0
    maxp = jnp.where(ok, maxp_ref[...], 0.0)
    sump = jnp.where(ok, sump_ref[...], 0.0)
    meanp = sump / jnp.maximum(cnt, 1.0)
    attn = jnp.where(ok, va_ref[...] / (den + 1e-16), 0.0)
    w = wfc_ref[...]
    out = jnp.dot(maxp, w[0:F], preferred_element_type=jnp.float32)
    out += jnp.dot(meanp, w[F:2 * F], preferred_element_type=jnp.float32)
    out += jnp.dot(sump, w[2 * F:3 * F], preferred_element_type=jnp.float32)
    out += jnp.dot(attn, w[3 * F:4 * F], preferred_element_type=jnp.float32)
    o_ref[...] = out + bfc_ref[...]


def _combine(maxp, sump, va, cnt, d, Wfc, bfcr):
    return pl.pallas_call(
        _comb_body,
        out_shape=jax.ShapeDtypeStruct((G, F), jnp.float32),
    )(maxp, sump, va, cnt, d, Wfc, bfcr)


def kernel(x, pos, batch, W1, b1, W2, b2, Wfc, bfc):
    del pos
    batch = batch.astype(jnp.int32)
    score2d, st = _score(x, W1, b1.reshape(1, F), W2.reshape(1, F),
                         b2.reshape(1, 1), batch.reshape(NB, 1, RB))
    score = score2d.reshape(N)
    counts = st.reshape(G)
    st_ex = jnp.cumsum(counts) - counts
    starts = jnp.concatenate(
        [st_ex.astype(jnp.int32), jnp.full((64,), N, jnp.int32)])
    maxp, sump, va, cnt, d = _sc_reduce(x, score, starts)
    out = _combine(maxp, sump, va, cnt, d, Wfc, bfc.reshape(1, F))
    return out[:1000]


# speedup vs baseline: 1.2515x; 1.2515x over previous
import functools

import jax
import jax.numpy as jnp
from jax import lax
from jax.experimental import pallas as pl
from jax.experimental.pallas import tpu as pltpu
from jax.experimental.pallas import tpu_sc as plsc

N = 100000
F = 128
G = 1024
SEG_PER_W = 32
C = 128
RB = 4000
NB = N // RB


def _score_body(x_ref, w1_ref, b1_ref, w2_ref, b2_ref, batch_ref,
                o_ref, st_ref, cacc):
    i = pl.program_id(0)

    @pl.when(i == 0)
    def _():
        cacc[...] = jnp.zeros_like(cacc)

    h = jnp.dot(x_ref[...], w1_ref[...], preferred_element_type=jnp.float32)
    h = h + b1_ref[...]
    h = jnp.maximum(h, 0.01 * h)
    s = jax.lax.dot_general(w2_ref[...], h, (((1,), (1,)), ((), ())),
                            preferred_element_type=jnp.float32)
    o_ref[0, 0, :] = s[0, :] + b2_ref[0, 0]

    bi = batch_ref[0, 0, :]
    hi_oh = jnp.equal(
        (bi // 128)[None, :],
        jax.lax.broadcasted_iota(jnp.int32, (8, RB), 0)).astype(jnp.float32)
    lo_oh = jnp.equal(
        (bi % 128)[:, None],
        jax.lax.broadcasted_iota(jnp.int32, (RB, F), 1)).astype(jnp.float32)
    cacc[...] += jnp.dot(hi_oh, lo_oh, preferred_element_type=jnp.float32)

    st_ref[...] = cacc[...].astype(jnp.int32)


def _score(x, W1, b1r, W2r, b2r, batch3d):
    return pl.pallas_call(
        _score_body,
        grid=(NB,),
        in_specs=[
            pl.BlockSpec((RB, F), lambda i: (i, 0)),
            pl.BlockSpec((F, F), lambda i: (0, 0)),
            pl.BlockSpec((1, F), lambda i: (0, 0)),
            pl.BlockSpec((1, F), lambda i: (0, 0)),
            pl.BlockSpec((1, 1), lambda i: (0, 0)),
            pl.BlockSpec((1, 1, RB), lambda i: (i, 0, 0)),
        ],
        out_specs=[
            pl.BlockSpec((1, 1, RB), lambda i: (i, 0, 0)),
            pl.BlockSpec((8, F), lambda i: (0, 0)),
        ],
        out_shape=[
            jax.ShapeDtypeStruct((NB, 1, RB), jnp.float32),
            jax.ShapeDtypeStruct((8, F), jnp.int32),
        ],
        scratch_shapes=[pltpu.VMEM((8, F), jnp.float32)],
    )(x, W1, b1r, W2r, b2r, batch3d)


def _sc_body(x_hbm, score_hbm, starts_hbm,
             maxp_hbm, sump_hbm, va_hbm, cnt_hbm, d_hbm,
             starts_v, xbuf, sbuf, stmax, stsum, stva, stcnt, std,
             semx0, semx1, sems0, sems1):
    wid = lax.axis_index("s") * 2 + lax.axis_index("c")
    seg_lo = wid * SEG_PER_W
    pltpu.sync_copy(starts_hbm.at[pl.ds(seg_lo, 64)], starts_v)

    neg_inf = jnp.full((16,), -jnp.inf, jnp.float32)
    zeros = jnp.zeros((16,), jnp.float32)
    init_acc = ((neg_inf,) * 8, (zeros,) * 8, (zeros,) * 8, neg_inf, zeros)

    sva = starts_v[pl.ds(0, 16)]
    rs = sva[0]
    svb = starts_v[pl.ds(32, 16)]
    re = svb[0]
    rsa = (rs // 8) * 8
    nch = jnp.where(re > rs, (re - rsa + C - 1) // C, 0)

    def start_chunk(c):
        bb = jnp.minimum(rsa + c * C, N - C)

        @pl.when((c & 1) == 0)
        def _():
            pltpu.async_copy(x_hbm.at[pl.ds(bb, C)],
                             xbuf.at[pl.ds(0, C)], semx0)
            pltpu.async_copy(score_hbm.at[pl.ds(bb, C)],
                             sbuf.at[pl.ds(0, C)], sems0)

        @pl.when((c & 1) == 1)
        def _():
            pltpu.async_copy(x_hbm.at[pl.ds(bb, C)],
                             xbuf.at[pl.ds(C, C)], semx1)
            pltpu.async_copy(score_hbm.at[pl.ds(bb, C)],
                             sbuf.at[pl.ds(C, C)], sems1)

    def wait_chunk(c):
        @pl.when((c & 1) == 0)
        def _():
            pltpu.make_async_copy(x_hbm.at[pl.ds(0, C)],
                                  xbuf.at[pl.ds(0, C)], semx0).wait()
            pltpu.make_async_copy(score_hbm.at[pl.ds(0, C)],
                                  sbuf.at[pl.ds(0, C)], sems0).wait()

        @pl.when((c & 1) == 1)
        def _():
            pltpu.make_async_copy(x_hbm.at[pl.ds(0, C)],
                                  xbuf.at[pl.ds(C, C)], semx1).wait()
            pltpu.make_async_copy(score_hbm.at[pl.ds(0, C)],
                                  sbuf.at[pl.ds(C, C)], sems1).wait()

    @pl.when(nch > 0)
    def _():
        start_chunk(jnp.int32(0))

    def row_body(r, acc):
        vmax, vsum, va, m, d = acc
        sv = jnp.full((16,), sbuf[pl.ds(r, 16)][0], jnp.float32)
        mn = jnp.maximum(m, sv)
        eo = jnp.exp(m - mn)
        en = jnp.exp(sv - mn)
        d2 = d * eo + en
        xs = [xbuf[r, pl.ds(16 * k, 16)] for k in range(8)]
        vmax2 = tuple(jnp.maximum(vmax[k], xs[k]) for k in range(8))
        vsum2 = tuple(vsum[k] + xs[k] for k in range(8))
        va2 = tuple(va[k] * eo + xs[k] * en for k in range(8))
        return (vmax2, vsum2, va2, mn, d2)

    def seg_body(s_rel, wdone):
        sv = starts_v[pl.ds(s_rel, 16)]
        s0 = sv[0]
        s1 = sv[1]
        cs0 = (s0 - rsa) // C
        cs1 = (s1 - 1 - rsa) // C
        ncs = jnp.where(s1 > s0, cs1 - cs0 + 1, 0)

        def piece(j, cy):
            acc, wdone = cy
            c = cs0 + j
            need = c >= wdone

            @pl.when(jnp.logical_and(need, c + 1 < nch))
            def _():
                start_chunk(c + 1)

            @pl.when(need)
            def _():
                wait_chunk(c)

            wdone = jnp.where(need, c + 1, wdone)
            base = rsa + c * C
            bb = jnp.minimum(base, N - C)
            off = (c & 1) * C - bb
            pg0 = jnp.maximum(s0, base)
            pg1 = jnp.minimum(s1, base + C)
            acc = lax.fori_loop(pg0 + off, pg1 + off, row_body, acc)
            return (acc, wdone)

        acc, wdone = lax.fori_loop(0, ncs, piece, (init_acc, wdone))
        vmax, vsum, va, m, d = acc
        cv = jnp.full((16,), (s1 - s0).astype(jnp.float32), jnp.float32)
        for k in range(8):
            stmax[pl.ds(s_rel * F + 16 * k, 16)] = vmax[k]
            stsum[pl.ds(s_rel * F + 16 * k, 16)] = vsum[k]
            stva[pl.ds(s_rel * F + 16 * k, 16)] = va[k]
        stcnt[pl.ds(s_rel * 16, 16)] = cv
        std[pl.ds(s_rel * 16, 16)] = d
        return wdone

    lax.fori_loop(0, SEG_PER_W, seg_body, 0)

    pltpu.sync_copy(stmax, maxp_hbm.at[pl.ds(seg_lo * F, SEG_PER_W * F)])
    pltpu.sync_copy(stsum, sump_hbm.at[pl.ds(seg_lo * F, SEG_PER_W * F)])
    pltpu.sync_copy(stva, va_hbm.at[pl.ds(seg_lo * F, SEG_PER_W * F)])
    pltpu.sync_copy(stcnt, cnt_hbm.at[pl.ds(seg_lo * 16, SEG_PER_W * 16)])
    pltpu.sync_copy(std, d_hbm.at[pl.ds(seg_lo * 16, SEG_PER_W * 16)])


def _sc_reduce(x, score, starts):
    mesh = plsc.VectorSubcoreMesh(core_axis_name="c", subcore_axis_name="s")
    f32 = jnp.float32
    fn = functools.partial(
        pl.kernel,
        mesh=mesh,
        out_type=[
            jax.ShapeDtypeStruct((G * F,), f32),
            jax.ShapeDtypeStruct((G * F,), f32),
            jax.ShapeDtypeStruct((G * F,), f32),
            jax.ShapeDtypeStruct((G * 16,), f32),
            jax.ShapeDtypeStruct((G * 16,), f32),
        ],
        scratch_types=[
            pltpu.VMEM((64,), jnp.int32),
            pltpu.VMEM((2 * C, F), f32),
            pltpu.VMEM((2 * C + 16,), f32),
            pltpu.VMEM((SEG_PER_W * F,), f32),
            pltpu.VMEM((SEG_PER_W * F,), f32),
            pltpu.VMEM((SEG_PER_W * F,), f32),
            pltpu.VMEM((SEG_PER_W * 16,), f32),
            pltpu.VMEM((SEG_PER_W * 16,), f32),
            pltpu.SemaphoreType.DMA,
            pltpu.SemaphoreType.DMA,
            pltpu.SemaphoreType.DMA,
            pltpu.SemaphoreType.DMA,
        ],
    )(_sc_body)
    maxp, sump, va, cnt, d = fn(x, score, starts)
    return (maxp.reshape(G, F), sump.reshape(G, F), va.reshape(G, F),
            cnt.reshape(G, 16), d.reshape(G, 16))


def _comb_body(maxp_ref, sump_ref, va_ref, cnt_ref, d_ref, wfc_ref, bfc_ref,
               o_ref):
    cnt = cnt_ref[...][:, 0:1]
    den = d_ref[...][:, 0:1]
    ok = cnt > 0
    maxp = jnp.where(ok, maxp_ref[...], 0.0)
    sump = jnp.where(ok, sump_ref[...], 0.0)
    meanp = sump / jnp.maximum(cnt, 1.0)
    attn = jnp.where(ok, va_ref[...] / (den + 1e-16), 0.0)
    w = wfc_ref[...]
    out = jnp.dot(maxp, w[0:F], preferred_element_type=jnp.float32)
    out += jnp.dot(meanp, w[F:2 * F], preferred_element_type=jnp.float32)
    out += jnp.dot(sump, w[2 * F:3 * F], preferred_element_type=jnp.float32)
    out += jnp.dot(attn, w[3 * F:4 * F], preferred_element_type=jnp.float32)
    o_ref[...] = out + bfc_ref[...]


def _combine(maxp, sump, va, cnt, d, Wfc, bfcr):
    return pl.pallas_call(
        _comb_body,
        out_shape=jax.ShapeDtypeStruct((G, F), jnp.float32),
    )(maxp, sump, va, cnt, d, Wfc, bfcr)


def kernel(x, pos, batch, W1, b1, W2, b2, Wfc, bfc):
    del pos
    batch = batch.astype(jnp.int32)
    score2d, st = _score(x, W1, b1.reshape(1, F), W2.reshape(1, F),
                         b2.reshape(1, 1), batch.reshape(NB, 1, RB))
    score = score2d.reshape(N)
    counts = st.reshape(G)
    st_ex = jnp.cumsum(counts) - counts
    starts = jnp.concatenate(
        [st_ex.astype(jnp.int32), jnp.full((64,), N, jnp.int32)])
    maxp, sump, va, cnt, d = _sc_reduce(x, score, starts)
    out = _combine(maxp, sump, va, cnt, d, Wfc, bfc.reshape(1, F))
    return out[:1000]


# speedup vs baseline: 1.3509x; 1.0794x over previous
import functools

import jax
import jax.numpy as jnp
from jax import lax
from jax.experimental import pallas as pl
from jax.experimental.pallas import tpu as pltpu
from jax.experimental.pallas import tpu_sc as plsc

N = 100000
F = 128
G = 1024
SEG_PER_W = 32
C = 128
RB = 10000
NB = N // RB


def _score_body(x_ref, w1_ref, b1_ref, w2_ref, b2_ref, batch_ref,
                o_ref, st_ref, cacc):
    i = pl.program_id(0)

    @pl.when(i == 0)
    def _():
        cacc[...] = jnp.zeros_like(cacc)

    h = jnp.dot(x_ref[...], w1_ref[...], preferred_element_type=jnp.float32)
    h = h + b1_ref[...]
    h = jnp.maximum(h, 0.01 * h)
    s = jax.lax.dot_general(w2_ref[...], h, (((1,), (1,)), ((), ())),
                            preferred_element_type=jnp.float32)
    o_ref[0, 0, :] = s[0, :] + b2_ref[0, 0]

    bi = batch_ref[0, 0, :]
    hi_oh = jnp.equal(
        (bi // 128)[None, :],
        jax.lax.broadcasted_iota(jnp.int32, (8, RB), 0)).astype(jnp.float32)
    lo_oh = jnp.equal(
        (bi % 128)[:, None],
        jax.lax.broadcasted_iota(jnp.int32, (RB, F), 1)).astype(jnp.float32)
    cacc[...] += jnp.dot(hi_oh, lo_oh, preferred_element_type=jnp.float32)

    st_ref[...] = cacc[...].astype(jnp.int32)


def _score(x, W1, b1r, W2r, b2r, batch3d):
    return pl.pallas_call(
        _score_body,
        grid=(NB,),
        in_specs=[
            pl.BlockSpec((RB, F), lambda i: (i, 0)),
            pl.BlockSpec((F, F), lambda i: (0, 0)),
            pl.BlockSpec((1, F), lambda i: (0, 0)),
            pl.BlockSpec((1, F), lambda i: (0, 0)),
            pl.BlockSpec((1, 1), lambda i: (0, 0)),
            pl.BlockSpec((1, 1, RB), lambda i: (i, 0, 0)),
        ],
        out_specs=[
            pl.BlockSpec((1, 1, RB), lambda i: (i, 0, 0)),
            pl.BlockSpec((8, F), lambda i: (0, 0)),
        ],
        out_shape=[
            jax.ShapeDtypeStruct((NB, 1, RB), jnp.float32),
            jax.ShapeDtypeStruct((8, F), jnp.int32),
        ],
        scratch_shapes=[pltpu.VMEM((8, F), jnp.float32)],
    )(x, W1, b1r, W2r, b2r, batch3d)


def _sc_body(x_hbm, score_hbm, starts_hbm,
             maxp_hbm, sump_hbm, va_hbm, cnt_hbm, d_hbm,
             starts_v, xbuf, sbuf, stmax, stsum, stva, stcnt, std,
             semx0, semx1, sems0, sems1):
    wid = lax.axis_index("s") * 2 + lax.axis_index("c")
    seg_lo = wid * SEG_PER_W
    pltpu.sync_copy(starts_hbm.at[pl.ds(seg_lo, 64)], starts_v)

    neg_inf = jnp.full((16,), -jnp.inf, jnp.float32)
    zeros = jnp.zeros((16,), jnp.float32)
    init_acc = ((neg_inf,) * 8, (zeros,) * 8, (zeros,) * 8, neg_inf, zeros)

    sva = starts_v[pl.ds(0, 16)]
    rs = sva[0]
    svb = starts_v[pl.ds(32, 16)]
    re = svb[0]
    rsa = (rs // 8) * 8
    nch = jnp.where(re > rs, (re - rsa + C - 1) // C, 0)

    def start_chunk(c):
        bb = jnp.minimum(rsa + c * C, N - C)

        @pl.when((c & 1) == 0)
        def _():
            pltpu.async_copy(x_hbm.at[pl.ds(bb, C)],
                             xbuf.at[pl.ds(0, C)], semx0)
            pltpu.async_copy(score_hbm.at[pl.ds(bb, C)],
                             sbuf.at[pl.ds(0, C)], sems0)

        @pl.when((c & 1) == 1)
        def _():
            pltpu.async_copy(x_hbm.at[pl.ds(bb, C)],
                             xbuf.at[pl.ds(C, C)], semx1)
            pltpu.async_copy(score_hbm.at[pl.ds(bb, C)],
                             sbuf.at[pl.ds(C, C)], sems1)

    def wait_chunk(c):
        @pl.when((c & 1) == 0)
        def _():
            pltpu.make_async_copy(x_hbm.at[pl.ds(0, C)],
                                  xbuf.at[pl.ds(0, C)], semx0).wait()
            pltpu.make_async_copy(score_hbm.at[pl.ds(0, C)],
                                  sbuf.at[pl.ds(0, C)], sems0).wait()

        @pl.when((c & 1) == 1)
        def _():
            pltpu.make_async_copy(x_hbm.at[pl.ds(0, C)],
                                  xbuf.at[pl.ds(C, C)], semx1).wait()
            pltpu.make_async_copy(score_hbm.at[pl.ds(0, C)],
                                  sbuf.at[pl.ds(C, C)], sems1).wait()

    @pl.when(nch > 0)
    def _():
        start_chunk(jnp.int32(0))

    def row_body(r, acc):
        vmax, vsum, va, m, d = acc
        sv = jnp.full((16,), sbuf[pl.ds(r, 16)][0], jnp.float32)
        mn = jnp.maximum(m, sv)
        eo = jnp.exp(m - mn)
        en = jnp.exp(sv - mn)
        d2 = d * eo + en
        xs = [xbuf[r, pl.ds(16 * k, 16)] for k in range(8)]
        vmax2 = tuple(jnp.maximum(vmax[k], xs[k]) for k in range(8))
        vsum2 = tuple(vsum[k] + xs[k] for k in range(8))
        va2 = tuple(va[k] * eo + xs[k] * en for k in range(8))
        return (vmax2, vsum2, va2, mn, d2)

    def seg_body(s_rel, wdone):
        sv = starts_v[pl.ds(s_rel, 16)]
        s0 = sv[0]
        s1 = sv[1]
        cs0 = (s0 - rsa) // C
        cs1 = (s1 - 1 - rsa) // C
        ncs = jnp.where(s1 > s0, cs1 - cs0 + 1, 0)

        def piece(j, cy):
            acc, wdone = cy
            c = cs0 + j
            need = c >= wdone

            @pl.when(jnp.logical_and(need, c + 1 < nch))
            def _():
                start_chunk(c + 1)

            @pl.when(need)
            def _():
                wait_chunk(c)

            wdone = jnp.where(need, c + 1, wdone)
            base = rsa + c * C
            bb = jnp.minimum(base, N - C)
            off = (c & 1) * C - bb
            pg0 = jnp.maximum(s0, base)
            pg1 = jnp.minimum(s1, base + C)
            acc = lax.fori_loop(pg0 + off, pg1 + off, row_body, acc)
            return (acc, wdone)

        acc, wdone = lax.fori_loop(0, ncs, piece, (init_acc, wdone))
        vmax, vsum, va, m, d = acc
        cv = jnp.full((16,), (s1 - s0).astype(jnp.float32), jnp.float32)
        for k in range(8):
            stmax[pl.ds(s_rel * F + 16 * k, 16)] = vmax[k]
            stsum[pl.ds(s_rel * F + 16 * k, 16)] = vsum[k]
            stva[pl.ds(s_rel * F + 16 * k, 16)] = va[k]
        stcnt[pl.ds(s_rel * 16, 16)] = cv
        std[pl.ds(s_rel * 16, 16)] = d
        return wdone

    lax.fori_loop(0, SEG_PER_W, seg_body, 0)

    pltpu.sync_copy(stmax, maxp_hbm.at[pl.ds(seg_lo * F, SEG_PER_W * F)])
    pltpu.sync_copy(stsum, sump_hbm.at[pl.ds(seg_lo * F, SEG_PER_W * F)])
    pltpu.sync_copy(stva, va_hbm.at[pl.ds(seg_lo * F, SEG_PER_W * F)])
    pltpu.sync_copy(stcnt, cnt_hbm.at[pl.ds(seg_lo * 16, SEG_PER_W * 16)])
    pltpu.sync_copy(std, d_hbm.at[pl.ds(seg_lo * 16, SEG_PER_W * 16)])


def _sc_reduce(x, score, starts):
    mesh = plsc.VectorSubcoreMesh(core_axis_name="c", subcore_axis_name="s")
    f32 = jnp.float32
    fn = functools.partial(
        pl.kernel,
        mesh=mesh,
        out_type=[
            jax.ShapeDtypeStruct((G * F,), f32),
            jax.ShapeDtypeStruct((G * F,), f32),
            jax.ShapeDtypeStruct((G * F,), f32),
            jax.ShapeDtypeStruct((G * 16,), f32),
            jax.ShapeDtypeStruct((G * 16,), f32),
        ],
        scratch_types=[
            pltpu.VMEM((64,), jnp.int32),
            pltpu.VMEM((2 * C, F), f32),
            pltpu.VMEM((2 * C + 16,), f32),
            pltpu.VMEM((SEG_PER_W * F,), f32),
            pltpu.VMEM((SEG_PER_W * F,), f32),
            pltpu.VMEM((SEG_PER_W * F,), f32),
            pltpu.VMEM((SEG_PER_W * 16,), f32),
            pltpu.VMEM((SEG_PER_W * 16,), f32),
            pltpu.SemaphoreType.DMA,
            pltpu.SemaphoreType.DMA,
            pltpu.SemaphoreType.DMA,
            pltpu.SemaphoreType.DMA,
        ],
    )(_sc_body)
    maxp, sump, va, cnt, d = fn(x, score, starts)
    return (maxp.reshape(G, F), sump.reshape(G, F), va.reshape(G, F),
            cnt.reshape(G, 16), d.reshape(G, 16))


def _comb_body(maxp_ref, sump_ref, va_ref, cnt_ref, d_ref, wfc_ref, bfc_ref,
               o_ref):
    cnt = cnt_ref[...][:, 0:1]
    den = d_ref[...][:, 0:1]
    ok = cnt > 0
    maxp = jnp.where(ok, maxp_ref[...], 0.0)
    sump = jnp.where(ok, sump_ref[...], 0.0)
    meanp = sump / jnp.maximum(cnt, 1.0)
    attn = jnp.where(ok, va_ref[...] / (den + 1e-16), 0.0)
    w = wfc_ref[...]
    out = jnp.dot(maxp, w[0:F], preferred_element_type=jnp.float32)
    out += jnp.dot(meanp, w[F:2 * F], preferred_element_type=jnp.float32)
    out += jnp.dot(sump, w[2 * F:3 * F], preferred_element_type=jnp.float32)
    out += jnp.dot(attn, w[3 * F:4 * F], preferred_element_type=jnp.float32)
    o_ref[...] = out + bfc_ref[...]


def _combine(maxp, sump, va, cnt, d, Wfc, bfcr):
    return pl.pallas_call(
        _comb_body,
        out_shape=jax.ShapeDtypeStruct((G, F), jnp.float32),
    )(maxp, sump, va, cnt, d, Wfc, bfcr)


def kernel(x, pos, batch, W1, b1, W2, b2, Wfc, bfc):
    del pos
    batch = batch.astype(jnp.int32)
    score2d, st = _score(x, W1, b1.reshape(1, F), W2.reshape(1, F),
                         b2.reshape(1, 1), batch.reshape(NB, 1, RB))
    score = score2d.reshape(N)
    counts = st.reshape(G)
    st_ex = jnp.cumsum(counts) - counts
    starts = jnp.concatenate(
        [st_ex.astype(jnp.int32), jnp.full((64,), N, jnp.int32)])
    maxp, sump, va, cnt, d = _sc_reduce(x, score, starts)
    out = _combine(maxp, sump, va, cnt, d, Wfc, bfc.reshape(1, F))
    return out[:1000]


# speedup vs baseline: 1.3788x; 1.0207x over previous
import functools

import jax
import jax.numpy as jnp
from jax import lax
from jax.experimental import pallas as pl
from jax.experimental.pallas import tpu as pltpu
from jax.experimental.pallas import tpu_sc as plsc

N = 100000
F = 128
G = 1024
SEG_PER_W = 32
C = 128
RB = 10000
NB = N // RB


def _score_body(x_ref, w1_ref, b1_ref, w2_ref, b2_ref, batch_ref,
                o_ref, st_ref, cacc):
    i = pl.program_id(0)

    @pl.when(i == 0)
    def _():
        cacc[...] = jnp.zeros_like(cacc)

    h = jnp.dot(x_ref[...], w1_ref[...], preferred_element_type=jnp.float32)
    h = h + b1_ref[...]
    h = jnp.maximum(h, 0.01 * h)
    s = jax.lax.dot_general(w2_ref[...], h, (((1,), (1,)), ((), ())),
                            preferred_element_type=jnp.float32)
    o_ref[0, 0, :] = s[0, :] + b2_ref[0, 0]

    bi = batch_ref[0, 0, :]
    hi_oh = jnp.equal(
        (bi // 128)[None, :],
        jax.lax.broadcasted_iota(jnp.int32, (8, RB), 0)).astype(jnp.float32)
    lo_oh = jnp.equal(
        (bi % 128)[:, None],
        jax.lax.broadcasted_iota(jnp.int32, (RB, F), 1)).astype(jnp.float32)
    cacc[...] += jnp.dot(hi_oh, lo_oh, preferred_element_type=jnp.float32)

    st_ref[...] = cacc[...].astype(jnp.int32)


def _score(x, W1, b1r, W2r, b2r, batch3d):
    return pl.pallas_call(
        _score_body,
        grid=(NB,),
        in_specs=[
            pl.BlockSpec((RB, F), lambda i: (i, 0)),
            pl.BlockSpec((F, F), lambda i: (0, 0)),
            pl.BlockSpec((1, F), lambda i: (0, 0)),
            pl.BlockSpec((1, F), lambda i: (0, 0)),
            pl.BlockSpec((1, 1), lambda i: (0, 0)),
            pl.BlockSpec((1, 1, RB), lambda i: (i, 0, 0)),
        ],
        out_specs=[
            pl.BlockSpec((1, 1, RB), lambda i: (i, 0, 0)),
            pl.BlockSpec((8, F), lambda i: (0, 0)),
        ],
        out_shape=[
            jax.ShapeDtypeStruct((NB, 1, RB), jnp.float32),
            jax.ShapeDtypeStruct((8, F), jnp.int32),
        ],
        scratch_shapes=[pltpu.VMEM((8, F), jnp.float32)],
    )(x, W1, b1r, W2r, b2r, batch3d)


def _sc_body(x_hbm, score_hbm, starts_hbm,
             maxp_hbm, sump_hbm, va_hbm, cnt_hbm, d_hbm,
             starts_v, xbuf, sbuf, stmax, stsum, stva, stcnt, std,
             semx0, semx1, sems0, sems1):
    wid = lax.axis_index("s") * 2 + lax.axis_index("c")
    seg_lo = wid * SEG_PER_W
    pltpu.sync_copy(starts_hbm.at[pl.ds(seg_lo, 64)], starts_v)

    neg_inf = jnp.full((16,), -jnp.inf, jnp.float32)
    zeros = jnp.zeros((16,), jnp.float32)
    init_acc = ((neg_inf,) * 8, (zeros,) * 8, (zeros,) * 8, neg_inf, zeros)

    sva = starts_v[pl.ds(0, 16)]
    rs = sva[0]
    svb = starts_v[pl.ds(32, 16)]
    re = svb[0]
    rsa = (rs // 8) * 8
    nch = jnp.where(re > rs, (re - rsa + C - 1) // C, 0)

    def start_chunk(c):
        bb = jnp.minimum(rsa + c * C, N - C)

        @pl.when((c & 1) == 0)
        def _():
            pltpu.async_copy(x_hbm.at[pl.ds(bb, C)],
                             xbuf.at[pl.ds(0, C)], semx0)
            pltpu.async_copy(score_hbm.at[pl.ds(bb, C)],
                             sbuf.at[pl.ds(0, C)], sems0)

        @pl.when((c & 1) == 1)
        def _():
            pltpu.async_copy(x_hbm.at[pl.ds(bb, C)],
                             xbuf.at[pl.ds(C, C)], semx1)
            pltpu.async_copy(score_hbm.at[pl.ds(bb, C)],
                             sbuf.at[pl.ds(C, C)], sems1)

    def wait_chunk(c):
        @pl.when((c & 1) == 0)
        def _():
            pltpu.make_async_copy(x_hbm.at[pl.ds(0, C)],
                                  xbuf.at[pl.ds(0, C)], semx0).wait()
            pltpu.make_async_copy(score_hbm.at[pl.ds(0, C)],
                                  sbuf.at[pl.ds(0, C)], sems0).wait()

        @pl.when((c & 1) == 1)
        def _():
            pltpu.make_async_copy(x_hbm.at[pl.ds(0, C)],
                                  xbuf.at[pl.ds(C, C)], semx1).wait()
            pltpu.make_async_copy(score_hbm.at[pl.ds(0, C)],
                                  sbuf.at[pl.ds(C, C)], sems1).wait()

    @pl.when(nch > 0)
    def _():
        start_chunk(jnp.int32(0))

    def row_body(r, acc):
        vmax, vsum, va, m, d = acc
        sv = jnp.full((16,), sbuf[pl.ds(r, 16)][0], jnp.float32)
        mn = jnp.maximum(m, sv)
        eo = jnp.exp(m - mn)
        en = jnp.exp(sv - mn)
        d2 = d * eo + en
        xs = [xbuf[r, pl.ds(16 * k, 16)] for k in range(8)]
        vmax2 = tuple(jnp.maximum(vmax[k], xs[k]) for k in range(8))
        vsum2 = tuple(vsum[k] + xs[k] for k in range(8))
        va2 = tuple(va[k] * eo + xs[k] * en for k in range(8))
        return (vmax2, vsum2, va2, mn, d2)

    def seg_body(s_rel, wdone):
        sv = starts_v[pl.ds(s_rel, 16)]
        s0 = sv[0]
        s1 = sv[1]
        cs0 = (s0 - rsa) // C
        cs1 = (s1 - 1 - rsa) // C
        ncs = jnp.where(s1 > s0, cs1 - cs0 + 1, 0)

        def piece(j, cy):
            acc, wdone = cy
            c = cs0 + j
            need = c >= wdone

            @pl.when(jnp.logical_and(need, c + 1 < nch))
            def _():
                start_chunk(c + 1)

            @pl.when(need)
            def _():
                wait_chunk(c)

            wdone = jnp.where(need, c + 1, wdone)
            base = rsa + c * C
            bb = jnp.minimum(base, N - C)
            off = (c & 1) * C - bb
            pg0 = jnp.maximum(s0, base)
            pg1 = jnp.minimum(s1, base + C)
            acc = lax.fori_loop(pg0 + off, pg1 + off, row_body, acc)
            return (acc, wdone)

        acc, wdone = lax.fori_loop(0, ncs, piece, (init_acc, wdone))
        vmax, vsum, va, m, d = acc
        cv = jnp.full((16,), (s1 - s0).astype(jnp.float32), jnp.float32)
        for k in range(8):
            stmax[pl.ds(s_rel * F + 16 * k, 16)] = vmax[k]
            stsum[pl.ds(s_rel * F + 16 * k, 16)] = vsum[k]
            stva[pl.ds(s_rel * F + 16 * k, 16)] = va[k]
        stcnt[pl.ds(s_rel * 16, 16)] = cv
        std[pl.ds(s_rel * 16, 16)] = d
        return wdone

    lax.fori_loop(0, SEG_PER_W, seg_body, 0)

    pltpu.sync_copy(stmax, maxp_hbm.at[pl.ds(seg_lo * F, SEG_PER_W * F)])
    pltpu.sync_copy(stsum, sump_hbm.at[pl.ds(seg_lo * F, SEG_PER_W * F)])
    pltpu.sync_copy(stva, va_hbm.at[pl.ds(seg_lo * F, SEG_PER_W * F)])
    pltpu.sync_copy(stcnt, cnt_hbm.at[pl.ds(seg_lo * 16, SEG_PER_W * 16)])
    pltpu.sync_copy(std, d_hbm.at[pl.ds(seg_lo * 16, SEG_PER_W * 16)])


def _sc_reduce(x, score, starts):
    mesh = plsc.VectorSubcoreMesh(core_axis_name="c", subcore_axis_name="s")
    f32 = jnp.float32
    fn = functools.partial(
        pl.kernel,
        mesh=mesh,
        out_type=[
            jax.ShapeDtypeStruct((G * F,), f32),
            jax.ShapeDtypeStruct((G * F,), f32),
            jax.ShapeDtypeStruct((G * F,), f32),
            jax.ShapeDtypeStruct((G * 16,), f32),
            jax.ShapeDtypeStruct((G * 16,), f32),
        ],
        scratch_types=[
            pltpu.VMEM((64,), jnp.int32),
            pltpu.VMEM((2 * C, F), f32),
            pltpu.VMEM((2 * C + 16,), f32),
            pltpu.VMEM((SEG_PER_W * F,), f32),
            pltpu.VMEM((SEG_PER_W * F,), f32),
            pltpu.VMEM((SEG_PER_W * F,), f32),
            pltpu.VMEM((SEG_PER_W * 16,), f32),
            pltpu.VMEM((SEG_PER_W * 16,), f32),
            pltpu.SemaphoreType.DMA,
            pltpu.SemaphoreType.DMA,
            pltpu.SemaphoreType.DMA,
            pltpu.SemaphoreType.DMA,
        ],
    )(_sc_body)
    maxp, sump, va, cnt, d = fn(x, score, starts)
    return (maxp, sump, va, cnt.reshape(G, 16), d.reshape(G, 16))


def _comb_body(maxp_ref, sump_ref, va_ref, cnt_ref, d_ref, wfc_ref, bfc_ref,
               o_ref):
    cnt = cnt_ref[...][:, 0:1]
    den = d_ref[...][:, 0:1]
    ok = cnt > 0
    maxp = jnp.where(ok, maxp_ref[...].reshape(G, F), 0.0)
    sump = jnp.where(ok, sump_ref[...].reshape(G, F), 0.0)
    meanp = sump / jnp.maximum(cnt, 1.0)
    attn = jnp.where(ok, va_ref[...].reshape(G, F) / (den + 1e-16), 0.0)
    w = wfc_ref[...]
    out = jnp.dot(maxp, w[0:F], preferred_element_type=jnp.float32)
    out += jnp.dot(meanp, w[F:2 * F], preferred_element_type=jnp.float32)
    out += jnp.dot(sump, w[2 * F:3 * F], preferred_element_type=jnp.float32)
    out += jnp.dot(attn, w[3 * F:4 * F], preferred_element_type=jnp.float32)
    o_ref[...] = (out + bfc_ref[...])[:1000]


def _combine(maxp, sump, va, cnt, d, Wfc, bfcr):
    return pl.pallas_call(
        _comb_body,
        out_shape=jax.ShapeDtypeStruct((1000, F), jnp.float32),
    )(maxp, sump, va, cnt, d, Wfc, bfcr)


def kernel(x, pos, batch, W1, b1, W2, b2, Wfc, bfc):
    del pos
    batch = batch.astype(jnp.int32)
    score2d, st = _score(x, W1, b1.reshape(1, F), W2.reshape(1, F),
                         b2.reshape(1, 1), batch.reshape(NB, 1, RB))
    score = score2d.reshape(N)
    counts = st.reshape(G)
    st_ex = jnp.cumsum(counts) - counts
    starts = jnp.concatenate(
        [st_ex.astype(jnp.int32), jnp.full((64,), N, jnp.int32)])
    maxp, sump, va, cnt, d = _sc_reduce(x, score, starts)
    return _combine(maxp, sump, va, cnt, d, Wfc, bfc.reshape(1, F))


# speedup vs baseline: 1.3807x; 1.0014x over previous
import functools

import jax
import jax.numpy as jnp
from jax import lax
from jax.experimental import pallas as pl
from jax.experimental.pallas import tpu as pltpu
from jax.experimental.pallas import tpu_sc as plsc

N = 100000
F = 128
G = 1024
SEG_PER_W = 32
C = 128
RB = 10000
NB = N // RB


def _score_body(x_ref, w1_ref, b1_ref, w2_ref, b2_ref, batch_ref,
                o_ref, st_ref, cacc):
    i = pl.program_id(0)

    @pl.when(i == 0)
    def _():
        cacc[...] = jnp.zeros_like(cacc)

    h = jnp.dot(x_ref[...], w1_ref[...], preferred_element_type=jnp.float32)
    h = h + b1_ref[...]
    h = jnp.maximum(h, 0.01 * h)
    s = jax.lax.dot_general(w2_ref[...], h, (((1,), (1,)), ((), ())),
                            preferred_element_type=jnp.float32)
    o_ref[0, 0, :] = s[0, :] + b2_ref[0, 0]

    bi = batch_ref[0, 0, :]
    hi_oh = jnp.equal(
        (bi // 128)[None, :],
        jax.lax.broadcasted_iota(jnp.int32, (8, RB), 0)).astype(jnp.float32)
    lo_oh = jnp.equal(
        (bi % 128)[:, None],
        jax.lax.broadcasted_iota(jnp.int32, (RB, F), 1)).astype(jnp.float32)
    cacc[...] += jnp.dot(hi_oh, lo_oh, preferred_element_type=jnp.float32)

    st_ref[...] = cacc[...].astype(jnp.int32)


def _score(x, W1, b1r, W2r, b2r, batch3d):
    return pl.pallas_call(
        _score_body,
        grid=(NB,),
        in_specs=[
            pl.BlockSpec((RB, F), lambda i: (i, 0)),
            pl.BlockSpec((F, F), lambda i: (0, 0)),
            pl.BlockSpec((1, F), lambda i: (0, 0)),
            pl.BlockSpec((1, F), lambda i: (0, 0)),
            pl.BlockSpec((1, 1), lambda i: (0, 0)),
            pl.BlockSpec((1, 1, RB), lambda i: (i, 0, 0)),
        ],
        out_specs=[
            pl.BlockSpec((1, 1, RB), lambda i: (i, 0, 0)),
            pl.BlockSpec((8, F), lambda i: (0, 0)),
        ],
        out_shape=[
            jax.ShapeDtypeStruct((NB, 1, RB), jnp.float32),
            jax.ShapeDtypeStruct((8, F), jnp.int32),
        ],
        scratch_shapes=[pltpu.VMEM((8, F), jnp.float32)],
    )(x, W1, b1r, W2r, b2r, batch3d)


def _sc_body(x_hbm, score_hbm, starts_hbm,
             maxp_hbm, sump_hbm, va_hbm, cnt_hbm, d_hbm,
             starts_v, xbuf, sbuf, stmax, stsum, stva, stcnt, std,
             semx0, semx1, sems0, sems1):
    wid = lax.axis_index("s") * 2 + lax.axis_index("c")
    seg_lo = wid * SEG_PER_W
    pltpu.sync_copy(starts_hbm.at[pl.ds(seg_lo, 64)], starts_v)

    neg_inf = jnp.full((16,), -jnp.inf, jnp.float32)
    zeros = jnp.zeros((16,), jnp.float32)
    init_acc = ((neg_inf,) * 8, (zeros,) * 8, (zeros,) * 8, neg_inf, zeros)

    sva = starts_v[pl.ds(0, 16)]
    rs = sva[0]
    svb = starts_v[pl.ds(32, 16)]
    re = svb[0]
    rsa = (rs // 8) * 8
    nch = jnp.where(re > rs, (re - rsa + C - 1) // C, 0)

    def start_chunk(c):
        bb = jnp.minimum(rsa + c * C, N - C)

        @pl.when((c & 1) == 0)
        def _():
            pltpu.async_copy(x_hbm.at[pl.ds(bb, C)],
                             xbuf.at[pl.ds(0, C)], semx0)
            pltpu.async_copy(score_hbm.at[pl.ds(bb, C)],
                             sbuf.at[pl.ds(0, C)], sems0)

        @pl.when((c & 1) == 1)
        def _():
            pltpu.async_copy(x_hbm.at[pl.ds(bb, C)],
                             xbuf.at[pl.ds(C, C)], semx1)
            pltpu.async_copy(score_hbm.at[pl.ds(bb, C)],
                             sbuf.at[pl.ds(C, C)], sems1)

    def wait_chunk(c):
        @pl.when((c & 1) == 0)
        def _():
            pltpu.make_async_copy(x_hbm.at[pl.ds(0, C)],
                                  xbuf.at[pl.ds(0, C)], semx0).wait()
            pltpu.make_async_copy(score_hbm.at[pl.ds(0, C)],
                                  sbuf.at[pl.ds(0, C)], sems0).wait()

        @pl.when((c & 1) == 1)
        def _():
            pltpu.make_async_copy(x_hbm.at[pl.ds(0, C)],
                                  xbuf.at[pl.ds(C, C)], semx1).wait()
            pltpu.make_async_copy(score_hbm.at[pl.ds(0, C)],
                                  sbuf.at[pl.ds(C, C)], sems1).wait()

    @pl.when(nch > 0)
    def _():
        start_chunk(jnp.int32(0))

    def row_body(r, acc):
        vmax, vsum, va, m, d = acc
        sv = jnp.full((16,), sbuf[pl.ds(r, 16)][0], jnp.float32)
        mn = jnp.maximum(m, sv)
        eo = jnp.exp(m - mn)
        en = jnp.exp(sv - mn)
        d2 = d * eo + en
        xs = [xbuf[r, pl.ds(16 * k, 16)] for k in range(8)]
        vmax2 = tuple(jnp.maximum(vmax[k], xs[k]) for k in range(8))
        vsum2 = tuple(vsum[k] + xs[k] for k in range(8))
        va2 = tuple(va[k] * eo + xs[k] * en for k in range(8))
        return (vmax2, vsum2, va2, mn, d2)

    def seg_body(s_rel, wdone):
        sv = starts_v[pl.ds(s_rel, 16)]
        s0 = sv[0]
        s1 = sv[1]
        cs0 = (s0 - rsa) // C
        cs1 = (s1 - 1 - rsa) // C
        ncs = jnp.where(s1 > s0, cs1 - cs0 + 1, 0)

        def piece(j, cy):
            acc, wdone = cy
            c = cs0 + j
            need = c >= wdone

            @pl.when(jnp.logical_and(need, c + 1 < nch))
            def _():
                start_chunk(c + 1)

            @pl.when(need)
            def _():
                wait_chunk(c)

            wdone = jnp.where(need, c + 1, wdone)
            base = rsa + c * C
            bb = jnp.minimum(base, N - C)
            off = (c & 1) * C - bb
            pg0 = jnp.maximum(s0, base)
            pg1 = jnp.minimum(s1, base + C)
            a0 = pg0 + off
            a1 = pg1 + off
            rem = (a1 - a0) & 1
            acc = lax.fori_loop(a0, a0 + rem, row_body, acc)

            def row2_body(i, acc):
                r = a0 + rem + 2 * i
                return row_body(r + 1, row_body(r, acc))

            acc = lax.fori_loop(0, (a1 - a0 - rem) // 2, row2_body, acc)
            return (acc, wdone)

        acc, wdone = lax.fori_loop(0, ncs, piece, (init_acc, wdone))
        vmax, vsum, va, m, d = acc
        cv = jnp.full((16,), (s1 - s0).astype(jnp.float32), jnp.float32)
        for k in range(8):
            stmax[pl.ds(s_rel * F + 16 * k, 16)] = vmax[k]
            stsum[pl.ds(s_rel * F + 16 * k, 16)] = vsum[k]
            stva[pl.ds(s_rel * F + 16 * k, 16)] = va[k]
        stcnt[pl.ds(s_rel * 16, 16)] = cv
        std[pl.ds(s_rel * 16, 16)] = d
        return wdone

    lax.fori_loop(0, SEG_PER_W, seg_body, 0)

    pltpu.sync_copy(stmax, maxp_hbm.at[pl.ds(seg_lo * F, SEG_PER_W * F)])
    pltpu.sync_copy(stsum, sump_hbm.at[pl.ds(seg_lo * F, SEG_PER_W * F)])
    pltpu.sync_copy(stva, va_hbm.at[pl.ds(seg_lo * F, SEG_PER_W * F)])
    pltpu.sync_copy(stcnt, cnt_hbm.at[pl.ds(seg_lo * 16, SEG_PER_W * 16)])
    pltpu.sync_copy(std, d_hbm.at[pl.ds(seg_lo * 16, SEG_PER_W * 16)])


def _sc_reduce(x, score, starts):
    mesh = plsc.VectorSubcoreMesh(core_axis_name="c", subcore_axis_name="s")
    f32 = jnp.float32
    fn = functools.partial(
        pl.kernel,
        mesh=mesh,
        out_type=[
            jax.ShapeDtypeStruct((G * F,), f32),
            jax.ShapeDtypeStruct((G * F,), f32),
            jax.ShapeDtypeStruct((G * F,), f32),
            jax.ShapeDtypeStruct((G * 16,), f32),
            jax.ShapeDtypeStruct((G * 16,), f32),
        ],
        scratch_types=[
            pltpu.VMEM((64,), jnp.int32),
            pltpu.VMEM((2 * C, F), f32),
            pltpu.VMEM((2 * C + 16,), f32),
            pltpu.VMEM((SEG_PER_W * F,), f32),
            pltpu.VMEM((SEG_PER_W * F,), f32),
            pltpu.VMEM((SEG_PER_W * F,), f32),
            pltpu.VMEM((SEG_PER_W * 16,), f32),
            pltpu.VMEM((SEG_PER_W * 16,), f32),
            pltpu.SemaphoreType.DMA,
            pltpu.SemaphoreType.DMA,
            pltpu.SemaphoreType.DMA,
            pltpu.SemaphoreType.DMA,
        ],
    )(_sc_body)
    maxp, sump, va, cnt, d = fn(x, score, starts)
    return (maxp, sump, va, cnt.reshape(G, 16), d.reshape(G, 16))


def _comb_body(maxp_ref, sump_ref, va_ref, cnt_ref, d_ref, wfc_ref, bfc_ref,
               o_ref):
    cnt = cnt_ref[...][:, 0:1]
    den = d_ref[...][:, 0:1]
    ok = cnt > 0
    maxp = jnp.where(ok, maxp_ref[...].reshape(G, F), 0.0)
    sump = jnp.where(ok, sump_ref[...].reshape(G, F), 0.0)
    meanp = sump / jnp.maximum(cnt, 1.0)
    attn = jnp.where(ok, va_ref[...].reshape(G, F) / (den + 1e-16), 0.0)
    w = wfc_ref[...]
    out = jnp.dot(maxp, w[0:F], preferred_element_type=jnp.float32)
    out += jnp.dot(meanp, w[F:2 * F], preferred_element_type=jnp.float32)
    out += jnp.dot(sump, w[2 * F:3 * F], preferred_element_type=jnp.float32)
    out += jnp.dot(attn, w[3 * F:4 * F], preferred_element_type=jnp.float32)
    o_ref[...] = (out + bfc_ref[...])[:1000]


def _combine(maxp, sump, va, cnt, d, Wfc, bfcr):
    return pl.pallas_call(
        _comb_body,
        out_shape=jax.ShapeDtypeStruct((1000, F), jnp.float32),
    )(maxp, sump, va, cnt, d, Wfc, bfcr)


def kernel(x, pos, batch, W1, b1, W2, b2, Wfc, bfc):
    del pos
    batch = batch.astype(jnp.int32)
    score2d, st = _score(x, W1, b1.reshape(1, F), W2.reshape(1, F),
                         b2.reshape(1, 1), batch.reshape(NB, 1, RB))
    score = score2d.reshape(N)
    counts = st.reshape(G)
    st_ex = jnp.cumsum(counts) - counts
    starts = jnp.concatenate(
        [st_ex.astype(jnp.int32), jnp.full((64,), N, jnp.int32)])
    maxp, sump, va, cnt, d = _sc_reduce(x, score, starts)
    return _combine(maxp, sump, va, cnt, d, Wfc, bfc.reshape(1, F))
